# baseline (speedup 1.0000x reference)
import functools

import jax
import jax.numpy as jnp
from jax import lax
from jax.experimental import pallas as pl
from jax.experimental.pallas import tpu as pltpu
from jax.experimental.pallas import tpu_sc as plsc

N = 10000
E = 320000
D = 128

NC = 2
NS = 16
NW = NC * NS
C = 64
NPASS = 10
PCHUNK = 16
EPT = NPASS * PCHUNK * C
E_PAD = NW * EPT
N_PAD = 10240
ROWS_PT = N_PAD // NS


def _mm_body(x_ref, w_ref, o_ref):
    o_ref[...] = jnp.dot(x_ref[...], w_ref[...],
                         preferred_element_type=jnp.float32)


def _matmul(x, w):
    MB = 1000
    return pl.pallas_call(
        _mm_body,
        grid=(N // MB,),
        in_specs=[pl.BlockSpec((MB, D), lambda i: (i, 0)),
                  pl.BlockSpec((D, D), lambda i: (0, 0))],
        out_specs=pl.BlockSpec((MB, D), lambda i: (i, 0)),
        out_shape=jax.ShapeDtypeStruct((N, D), jnp.float32),
    )(x, w)


def _comb_body(p_ref, b_ref, o_ref):
    o_ref[...] = p_ref[0] + p_ref[1] + b_ref[0:1]


def _combine(partials, bias):
    MB = 1000
    bias8 = jnp.broadcast_to(bias.reshape(1, D), (8, D))
    return pl.pallas_call(
        _comb_body,
        grid=(N // MB,),
        in_specs=[pl.BlockSpec((2, MB, D), lambda i: (0, i, 0)),
                  pl.BlockSpec((8, D), lambda i: (0, 0))],
        out_specs=pl.BlockSpec((MB, D), lambda i: (i, 0)),
        out_shape=jax.ShapeDtypeStruct((N, D), jnp.float32),
    )(partials, bias8)


_mesh = plsc.VectorSubcoreMesh(core_axis_name="c", subcore_axis_name="s")


@functools.partial(
    pl.kernel,
    out_type=jax.ShapeDtypeStruct((NC, N_PAD, D), jnp.float32),
    mesh=_mesh,
    scratch_types=[
        pltpu.VMEM_SHARED((N_PAD, D), jnp.float32),
        pltpu.VMEM((PCHUNK, 2, C), jnp.int32),
        pltpu.VMEM((PCHUNK, C * 16), jnp.float32),
        pltpu.VMEM((C, D), jnp.float32),
        pltpu.VMEM((C, D), jnp.float32),
        pltpu.SemaphoreType.DMA,
        pltpu.SemaphoreType.DMA,
        pltpu.SemaphoreType.DMA,
        pltpu.SemaphoreType.DMA,
    ],
)
def _sc_edges(sup, rc, wx, out, acc, rc_v, wx_v,
              rows0, rows1, g0, g1, s0, s1):
    c = lax.axis_index("c")
    s = lax.axis_index("s")
    wid = s * NC + c

    zero16 = jnp.zeros((16,), jnp.float32)

    def _zrow(r, _):
        for j in range(D // 16):
            rows0[r, pl.ds(j * 16, 16)] = zero16
        return 0

    lax.fori_loop(0, C, _zrow, 0)
    for k in range(ROWS_PT // C):
        pltpu.sync_copy(rows0, acc.at[pl.ds(s * ROWS_PT + k * C, C)])
    plsc.subcore_barrier()

    def _scale(buf, k):
        @plsc.parallel_loop(0, C, 1, unroll=4)
        def _edge(e):
            wsp = wx_v[k, pl.ds(e * 16, 16)]
            for j in range(D // 16):
                sl = pl.ds(j * 16, 16)
                buf[e, sl] = buf[e, sl] * wsp

    def _pair(i, _):
        k0 = 2 * i
        k1 = 2 * i + 1
        pltpu.make_async_copy(sup.at[rc_v.at[k0, 0]], rows0, g0).wait()
        _scale(rows0, k0)

        @pl.when(i > 0)
        def _():
            pltpu.make_async_copy(rows1, acc.at[rc_v.at[k1, 1]], s1).wait()

        pltpu.async_copy(sup.at[rc_v.at[k1, 0]], rows1, g1)
        pltpu.async_copy(rows0, acc.at[rc_v.at[k0, 1]], s0, add=True)
        pltpu.make_async_copy(sup.at[rc_v.at[k1, 0]], rows1, g1).wait()
        _scale(rows1, k1)
        pltpu.make_async_copy(rows0, acc.at[rc_v.at[k0, 1]], s0).wait()

        @pl.when(k0 + 2 < PCHUNK)
        def _():
            pltpu.async_copy(sup.at[rc_v.at[k0 + 2, 0]], rows0, g0)

        pltpu.async_copy(rows1, acc.at[rc_v.at[k1, 1]], s1, add=True)
        return 0

    for p in range(NPASS):
        pltpu.sync_copy(rc.at[wid, p], rc_v)
        pltpu.sync_copy(wx.at[wid, p], wx_v)
        pltpu.async_copy(sup.at[rc_v.at[0, 0]], rows0, g0)
        lax.fori_loop(0, PCHUNK // 2, _pair, 0)
        pltpu.make_async_copy(rows1, acc.at[rc_v.at[PCHUNK - 1, 1]],
                              s1).wait()

    plsc.subcore_barrier()

    pltpu.sync_copy(acc.at[pl.ds(s * ROWS_PT, ROWS_PT)],
                    out.at[c, pl.ds(s * ROWS_PT, ROWS_PT)])


def kernel(x, edge_index, edge_weight, weight, bias):
    npad = E_PAD - E
    row = jnp.concatenate(
        [edge_index[0].astype(jnp.int32), jnp.zeros((npad,), jnp.int32)])
    col = jnp.concatenate(
        [edge_index[1].astype(jnp.int32), jnp.zeros((npad,), jnp.int32)])
    ewp = jnp.concatenate([edge_weight, jnp.zeros((npad,), jnp.float32)])
    row = row.reshape(NW, NPASS, PCHUNK, C)
    col = col.reshape(NW, NPASS, PCHUNK, C)
    rc = jnp.stack([row, col], axis=3)
    wx = jnp.broadcast_to(ewp[:, None], (E_PAD, 16))
    wx = wx.reshape(NW, NPASS, PCHUNK, C * 16)
    support = _matmul(x, weight)
    partials = _sc_edges(support, rc, wx)
    return _combine(partials, bias)


# speedup vs baseline: 1.2083x; 1.2083x over previous
import functools

import jax
import jax.numpy as jnp
from jax import lax
from jax.experimental import pallas as pl
from jax.experimental.pallas import tpu as pltpu
from jax.experimental.pallas import tpu_sc as plsc

N = 10000
E = 320000
D = 128

NC = 2
NS = 16
NW = NC * NS
C = 80
NPASS = 8
PCHUNK = 16
EPT = NPASS * PCHUNK * C
E_PAD = NW * EPT
N_PAD = 10240
ROWS_PT = N_PAD // NS


def _mm_body(x_ref, w_ref, o_ref):
    o_ref[...] = jnp.dot(x_ref[...], w_ref[...],
                         preferred_element_type=jnp.float32)


def _matmul(x, w):
    MB = 1000
    return pl.pallas_call(
        _mm_body,
        grid=(N // MB,),
        in_specs=[pl.BlockSpec((MB, D), lambda i: (i, 0)),
                  pl.BlockSpec((D, D), lambda i: (0, 0))],
        out_specs=pl.BlockSpec((MB, D), lambda i: (i, 0)),
        out_shape=jax.ShapeDtypeStruct((N, D), jnp.float32),
    )(x, w)


def _comb_body(p_ref, b_ref, o_ref):
    o_ref[...] = p_ref[0] + p_ref[1] + b_ref[0:1]


def _combine(partials, bias):
    MB = 1000
    bias8 = jnp.broadcast_to(bias.reshape(1, D), (8, D))
    return pl.pallas_call(
        _comb_body,
        grid=(N // MB,),
        in_specs=[pl.BlockSpec((2, MB, D), lambda i: (0, i, 0)),
                  pl.BlockSpec((8, D), lambda i: (0, 0))],
        out_specs=pl.BlockSpec((MB, D), lambda i: (i, 0)),
        out_shape=jax.ShapeDtypeStruct((N, D), jnp.float32),
    )(partials, bias8)


_mesh = plsc.VectorSubcoreMesh(core_axis_name="c", subcore_axis_name="s")


@functools.partial(
    pl.kernel,
    out_type=jax.ShapeDtypeStruct((NC, N_PAD, D), jnp.float32),
    mesh=_mesh,
    scratch_types=[
        pltpu.VMEM_SHARED((N_PAD, D), jnp.float32),
        pltpu.VMEM((PCHUNK, 2, C), jnp.int32),
        pltpu.VMEM((PCHUNK, C), jnp.float32),
        pltpu.VMEM((2, C, D), jnp.float32),
        pltpu.SemaphoreType.DMA((2,)),
        pltpu.SemaphoreType.DMA((2,)),
    ],
)
def _sc_edges(sup, rc, ew, out, acc, rc_v, w_v, rows_v, gsem, ssem):
    c = lax.axis_index("c")
    s = lax.axis_index("s")
    wid = s * NC + c

    zero16 = jnp.zeros((16,), jnp.float32)

    def _zrow(r, _):
        for j in range(D // 16):
            rows_v[0, r, pl.ds(j * 16, 16)] = zero16
        return 0

    lax.fori_loop(0, C, _zrow, 0)
    for k in range(ROWS_PT // C):
        pltpu.sync_copy(rows_v.at[0], acc.at[pl.ds(s * ROWS_PT + k * C, C)])
    plsc.subcore_barrier()

    def _chunk(k, _):
        b = lax.rem(k, 2)
        nb = 1 - b
        buf = rows_v.at[b]
        pltpu.make_async_copy(sup.at[rc_v.at[k, 0]], buf, gsem.at[b]).wait()

        @pl.when(k >= 1)
        def _():
            pltpu.make_async_copy(rows_v.at[nb], acc.at[rc_v.at[k - 1, 1]],
                                  ssem.at[nb]).wait()

        @pl.when(k + 1 < PCHUNK)
        def _():
            pltpu.async_copy(sup.at[rc_v.at[k + 1, 0]], rows_v.at[nb],
                             gsem.at[nb])

        for g in range(C // 16):
            wvec = w_v[k, pl.ds(g * 16, 16)]
            for t in range(16):
                e = g * 16 + t
                w_e = wvec[t]
                for j in range(D // 16):
                    sl = pl.ds(j * 16, 16)
                    buf[e, sl] = buf[e, sl] * w_e

        pltpu.async_copy(buf, acc.at[rc_v.at[k, 1]], ssem.at[b], add=True)
        return 0

    for p in range(NPASS):
        pltpu.sync_copy(rc.at[wid, p], rc_v)
        pltpu.sync_copy(ew.at[wid, p], w_v)
        pltpu.async_copy(sup.at[rc_v.at[0, 0]], rows_v.at[0], gsem.at[0])
        lax.fori_loop(0, PCHUNK, _chunk, 0)
        lastb = (PCHUNK - 1) % 2
        pltpu.make_async_copy(rows_v.at[lastb],
                              acc.at[rc_v.at[PCHUNK - 1, 1]],
                              ssem.at[lastb]).wait()

    plsc.subcore_barrier()

    pltpu.sync_copy(acc.at[pl.ds(s * ROWS_PT, ROWS_PT)],
                    out.at[c, pl.ds(s * ROWS_PT, ROWS_PT)])


def kernel(x, edge_index, edge_weight, weight, bias):
    npad = E_PAD - E
    row = jnp.concatenate(
        [edge_index[0].astype(jnp.int32), jnp.zeros((npad,), jnp.int32)])
    col = jnp.concatenate(
        [edge_index[1].astype(jnp.int32), jnp.zeros((npad,), jnp.int32)])
    ewp = jnp.concatenate([edge_weight, jnp.zeros((npad,), jnp.float32)])
    row = row.reshape(NW, NPASS, PCHUNK, C)
    col = col.reshape(NW, NPASS, PCHUNK, C)
    rc = jnp.stack([row, col], axis=3)
    ew = ewp.reshape(NW, NPASS, PCHUNK, C)
    support = _matmul(x, weight)
    partials = _sc_edges(support, rc, ew)
    return _combine(partials, bias)


# speedup vs baseline: 2.2919x; 1.8967x over previous
import functools

import jax
import jax.numpy as jnp
from jax import lax
from jax.experimental import pallas as pl
from jax.experimental.pallas import tpu as pltpu
from jax.experimental.pallas import tpu_sc as plsc

N = 10000
E = 320000
D = 128

NC = 2
NS = 16
NW = NC * NS
C = 80
EPT = E // NW
NPASS = 5
PCHUNK = EPT // C // NPASS
N_PAD = 10240
ROWS_PT = N_PAD // NS


def _mm_body(x_ref, w_ref, o_ref):
    o_ref[...] = jnp.dot(x_ref[...], w_ref[...],
                         preferred_element_type=jnp.float32)


def _matmul(x, w):
    MB = 1000
    return pl.pallas_call(
        _mm_body,
        grid=(N // MB,),
        in_specs=[pl.BlockSpec((MB, D), lambda i: (i, 0)),
                  pl.BlockSpec((D, D), lambda i: (0, 0))],
        out_specs=pl.BlockSpec((MB, D), lambda i: (i, 0)),
        out_shape=jax.ShapeDtypeStruct((N, D), jnp.float32),
    )(x, w)


def _comb_body(p_ref, b_ref, o_ref):
    o_ref[...] = p_ref[0] + p_ref[1] + b_ref[0:1]


def _combine(partials, bias):
    MB = 1000
    bias8 = jnp.broadcast_to(bias.reshape(1, D), (8, D))
    return pl.pallas_call(
        _comb_body,
        grid=(N // MB,),
        in_specs=[pl.BlockSpec((2, MB, D), lambda i: (0, i, 0)),
                  pl.BlockSpec((8, D), lambda i: (0, 0))],
        out_specs=pl.BlockSpec((MB, D), lambda i: (i, 0)),
        out_shape=jax.ShapeDtypeStruct((N, D), jnp.float32),
    )(partials, bias8)


_mesh = plsc.VectorSubcoreMesh(core_axis_name="c", subcore_axis_name="s")


@functools.partial(
    pl.kernel,
    out_type=jax.ShapeDtypeStruct((NC, N_PAD, D), jnp.float32),
    mesh=_mesh,
    scratch_types=[
        pltpu.VMEM_SHARED((N_PAD, D), jnp.float32),
        pltpu.VMEM((PCHUNK, 2, C), jnp.int32),
        pltpu.VMEM((PCHUNK, C), jnp.float32),
        pltpu.VMEM((C, D), jnp.float32),
        pltpu.SemaphoreType.DMA,
    ],
)
def _sc_edges(sup, rc, ew, out, acc, rc_v, w_v, rows_v, sem):
    c = lax.axis_index("c")
    s = lax.axis_index("s")
    wid = s * NC + c

    zero16 = jnp.zeros((16,), jnp.float32)

    def _zrow(r, _):
        for j in range(D // 16):
            rows_v[r, pl.ds(j * 16, 16)] = zero16
        return 0

    lax.fori_loop(0, C, _zrow, 0)
    for k in range(ROWS_PT // C):
        pltpu.sync_copy(rows_v, acc.at[pl.ds(s * ROWS_PT + k * C, C)])
    plsc.subcore_barrier()

    def _chunk(k, _):
        pltpu.async_copy(sup.at[rc_v.at[k, 0]], rows_v, sem).wait()
        for g in range(C // 16):
            wvec = w_v[k, pl.ds(g * 16, 16)]
            for t in range(16):
                e = g * 16 + t
                w_e = wvec[t]
                for j in range(D // 16):
                    sl = pl.ds(j * 16, 16)
                    rows_v[e, sl] = rows_v[e, sl] * w_e
        pltpu.sync_copy(rows_v, acc.at[rc_v.at[k, 1]], add=True)
        return 0

    for p in range(NPASS):
        pltpu.sync_copy(rc.at[wid, p], rc_v)
        pltpu.sync_copy(ew.at[wid, p], w_v)
        lax.fori_loop(0, PCHUNK, _chunk, 0)
    plsc.subcore_barrier()

    pltpu.sync_copy(acc.at[pl.ds(s * ROWS_PT, ROWS_PT)],
                    out.at[c, pl.ds(s * ROWS_PT, ROWS_PT)])


def kernel(x, edge_index, edge_weight, weight, bias):
    row = edge_index[0].astype(jnp.int32).reshape(NW, NPASS, PCHUNK, C)
    col = edge_index[1].astype(jnp.int32).reshape(NW, NPASS, PCHUNK, C)
    rc = jnp.stack([row, col], axis=3)
    ew = edge_weight.reshape(NW, NPASS, PCHUNK, C)
    support = _matmul(x, weight)
    partials = _sc_edges(support, rc, ew)
    return _combine(partials, bias)
